# baseline (speedup 1.0000x reference)
import functools
import jax
import jax.numpy as jnp
from jax import lax
from jax.experimental import pallas as pl
from jax.experimental.pallas import tpu as pltpu
from jax.experimental.pallas import tpu_sc as plsc

V = 1000000
D = 64
B = 4096
S = 200

NC = 2
NS = 16
NW = NC * NS

CBLK = 128
NBLK = (V + CBLK - 1) // CBLK
FULL_ROUNDS = 244
TAIL_C = NBLK - 1

B_PER_W = B // NW
HALF = S // 2

_mesh = plsc.VectorSubcoreMesh(core_axis_name="c", subcore_axis_name="s")


def _build_transpose():
    @functools.partial(
        pl.kernel,
        mesh=_mesh,
        out_type=jax.ShapeDtypeStruct((V, 128), jnp.float32),
        scratch_types=[
            pltpu.VMEM((D, CBLK), jnp.float32),
            pltpu.VMEM((D, CBLK), jnp.float32),
            pltpu.VMEM((CBLK, 128), jnp.float32),
            pltpu.VMEM((CBLK, 128), jnp.float32),
            pltpu.SemaphoreType.DMA,
            pltpu.SemaphoreType.DMA,
            pltpu.SemaphoreType.DMA,
            pltpu.SemaphoreType.DMA,
        ],
        compiler_params=pltpu.CompilerParams(
            use_tc_tiling_on_sc=True, needs_layout_passes=False
        ),
    )
    def transpose_kernel(tableT_hbm, tail_hbm, scratch_hbm,
                         blk0, blk1, tb0, tb1, l0, l1, w0, w1):
        wid = lax.axis_index("s") * NC + lax.axis_index("c")
        blks = (blk0, blk1)
        tbs = (tb0, tb1)
        lsems = (l0, l1)
        wsems = (w0, w1)
        iota16 = lax.iota(jnp.int32, 16)

        def fire_loads(k, b):
            c = wid + NW * k
            for dt in range(D // 8):
                pltpu.async_copy(
                    tableT_hbm.at[pl.ds(dt * 8, 8), pl.ds(c * CBLK, CBLK)],
                    blks[b].at[pl.ds(dt * 8, 8)],
                    lsems[b],
                )

        def drain_loads(b):
            pltpu.make_async_copy(
                tableT_hbm.at[:, pl.ds(0, 128)], blks[b], lsems[b]
            ).wait()

        def transpose_block(blk, tb):
            def jloop(j, carry):
                jvec = jnp.full((16,), j, jnp.int32)
                for dd in range(D // 16):
                    piece = plsc.load_gather(blk, [dd * 16 + iota16, jvec])
                    tb[j, pl.ds(dd * 16, 16)] = piece
                return carry
            lax.fori_loop(0, CBLK, jloop, 0)

        def fire_write(k, b):
            c = wid + NW * k
            pltpu.async_copy(
                tbs[b], scratch_hbm.at[pl.ds(c * CBLK, CBLK)], wsems[b]
            )

        def wait_write(k, b):
            c = wid + NW * k
            pltpu.make_async_copy(
                tbs[b], scratch_hbm.at[pl.ds(c * CBLK, CBLK)], wsems[b]
            ).wait()

        def step(k, b):
            drain_loads(b)

            @pl.when(k <= FULL_ROUNDS - 2)
            def _():
                fire_loads(k + 1, 1 - b)

            @pl.when(k >= 2)
            def _():
                wait_write(k - 2, b)

            transpose_block(blks[b], tbs[b])
            fire_write(k, b)

        fire_loads(0, 0)

        def loop_body(i, carry):
            step(2 * i, 0)
            step(2 * i + 1, 1)
            return carry

        lax.fori_loop(0, FULL_ROUNDS // 2, loop_body, 0)
        wait_write(FULL_ROUNDS - 2, 0)
        wait_write(FULL_ROUNDS - 1, 1)

        c_last = wid + NW * FULL_ROUNDS

        @pl.when(c_last < TAIL_C)
        def _():
            fire_loads(FULL_ROUNDS, 0)
            drain_loads(0)
            transpose_block(blk0, tb0)
            fire_write(FULL_ROUNDS, 0)
            wait_write(FULL_ROUNDS, 0)

        @pl.when(c_last == TAIL_C)
        def _():
            pltpu.sync_copy(tail_hbm, tb1.at[pl.ds(0, D)])
            pltpu.sync_copy(
                tb1.at[pl.ds(0, D)], scratch_hbm.at[pl.ds(TAIL_C * CBLK, D)]
            )

    return transpose_kernel


def _build_gather():
    @functools.partial(
        pl.kernel,
        mesh=_mesh,
        out_type=jax.ShapeDtypeStruct((B, S, D), jnp.float32),
        scratch_types=[
            pltpu.VMEM((B_PER_W * 2, HALF), jnp.int32),
            pltpu.VMEM((S, 128), jnp.float32),
            pltpu.VMEM((S, 128), jnp.float32),
            pltpu.SemaphoreType.DMA,
            pltpu.SemaphoreType.DMA,
            pltpu.SemaphoreType.DMA,
            pltpu.SemaphoreType.DMA,
        ],
        compiler_params=pltpu.CompilerParams(use_tc_tiling_on_sc=False),
    )
    def gather_kernel(scratch_hbm, ids_hbm, out_hbm,
                      idsv, rows0, rows1, g0, g1, w0, w1):
        wid = lax.axis_index("s") * NC + lax.axis_index("c")
        b0 = wid * B_PER_W
        rows = (rows0, rows1)
        gsems = (g0, g1)
        wsems = (w0, w1)

        def fire_gather(i, b):
            for h in range(2):
                pltpu.async_copy(
                    scratch_hbm.at[idsv.at[2 * i + h]],
                    rows[b].at[pl.ds(h * HALF, HALF)],
                    gsems[b],
                )

        def wait_gather(i, b):
            for h in range(2):
                pltpu.make_async_copy(
                    scratch_hbm.at[idsv.at[2 * i + h]],
                    rows[b].at[pl.ds(h * HALF, HALF)],
                    gsems[b],
                ).wait()

        def fire_out(i, b):
            pltpu.async_copy(
                rows[b].at[:, pl.ds(0, D)], out_hbm.at[b0 + i], wsems[b]
            )

        def wait_out(i, b):
            pltpu.make_async_copy(
                rows[b].at[:, pl.ds(0, D)], out_hbm.at[b0 + i], wsems[b]
            ).wait()

        pltpu.sync_copy(ids_hbm.at[pl.ds(wid * B_PER_W * 2, B_PER_W * 2)], idsv)

        fire_gather(0, 0)

        def step(i, b):
            @pl.when(i >= 1)
            def _():
                wait_out(i - 1, 1 - b)

            @pl.when(i <= B_PER_W - 2)
            def _():
                fire_gather(i + 1, 1 - b)

            wait_gather(i, b)
            fire_out(i, b)

        def loop_body(j, carry):
            step(2 * j, 0)
            step(2 * j + 1, 1)
            return carry

        lax.fori_loop(0, B_PER_W // 2, loop_body, 0)
        wait_out(B_PER_W - 1, 1)

    return gather_kernel


_transpose = _build_transpose()
_gather = _build_gather()


@jax.jit
def kernel(input_ids, table):
    tableT = jnp.transpose(table)
    ids2d = input_ids.astype(jnp.int32).reshape(B * 2, HALF)
    tail = jnp.pad(table[TAIL_C * CBLK :], ((0, 0), (0, 128 - D)))
    scratch = _transpose(tableT, tail)
    return _gather(scratch, ids2d)


# speedup vs baseline: 1.0015x; 1.0015x over previous
import functools
import jax
import jax.numpy as jnp
from jax import lax
from jax.experimental import pallas as pl
from jax.experimental.pallas import tpu as pltpu
from jax.experimental.pallas import tpu_sc as plsc

V = 1000000
D = 64
B = 4096
S = 200

NC = 2
NS = 16
NW = NC * NS

CBLK = 128
NBLK = (V + CBLK - 1) // CBLK
FULL_ROUNDS = 244
TAIL_C = NBLK - 1

B_PER_W = B // NW
HALF = S // 2

_mesh = plsc.VectorSubcoreMesh(core_axis_name="c", subcore_axis_name="s")


def _build_transpose():
    @functools.partial(
        pl.kernel,
        mesh=_mesh,
        out_type=jax.ShapeDtypeStruct((V, 128), jnp.float32),
        scratch_types=[
            pltpu.VMEM((D, CBLK), jnp.float32),
            pltpu.VMEM((D, CBLK), jnp.float32),
            pltpu.VMEM((CBLK, 128), jnp.float32),
            pltpu.VMEM((CBLK, 128), jnp.float32),
            pltpu.SemaphoreType.DMA,
            pltpu.SemaphoreType.DMA,
            pltpu.SemaphoreType.DMA,
            pltpu.SemaphoreType.DMA,
        ],
        compiler_params=pltpu.CompilerParams(
            use_tc_tiling_on_sc=True, needs_layout_passes=False
        ),
    )
    def transpose_kernel(tableT_hbm, tail_hbm, scratch_hbm,
                         blk0, blk1, tb0, tb1, l0, l1, w0, w1):
        wid = lax.axis_index("s") * NC + lax.axis_index("c")
        blks = (blk0, blk1)
        tbs = (tb0, tb1)
        lsems = (l0, l1)
        wsems = (w0, w1)
        iota16 = lax.iota(jnp.int32, 16)

        def fire_loads(k, b):
            c = wid + NW * k
            for dt in range(D // 8):
                pltpu.async_copy(
                    tableT_hbm.at[pl.ds(dt * 8, 8), pl.ds(c * CBLK, CBLK)],
                    blks[b].at[pl.ds(dt * 8, 8)],
                    lsems[b],
                )

        def drain_loads(b):
            pltpu.make_async_copy(
                tableT_hbm.at[:, pl.ds(0, 128)], blks[b], lsems[b]
            ).wait()

        dvecs = tuple(dd * 16 + iota16 for dd in range(D // 16))

        def transpose_block(blk, tb):
            def jloop(jj, carry):
                j0 = jj * 8
                for ju in range(8):
                    jvec = jnp.full((16,), j0 + ju, jnp.int32)
                    for dd in range(D // 16):
                        piece = plsc.load_gather(blk, [dvecs[dd], jvec])
                        tb[j0 + ju, pl.ds(dd * 16, 16)] = piece
                return carry
            lax.fori_loop(0, CBLK // 8, jloop, 0)

        def fire_write(k, b):
            c = wid + NW * k
            pltpu.async_copy(
                tbs[b], scratch_hbm.at[pl.ds(c * CBLK, CBLK)], wsems[b]
            )

        def wait_write(k, b):
            c = wid + NW * k
            pltpu.make_async_copy(
                tbs[b], scratch_hbm.at[pl.ds(c * CBLK, CBLK)], wsems[b]
            ).wait()

        def step(k, b):
            drain_loads(b)

            @pl.when(k <= FULL_ROUNDS - 2)
            def _():
                fire_loads(k + 1, 1 - b)

            @pl.when(k >= 2)
            def _():
                wait_write(k - 2, b)

            transpose_block(blks[b], tbs[b])
            fire_write(k, b)

        fire_loads(0, 0)

        def loop_body(i, carry):
            step(2 * i, 0)
            step(2 * i + 1, 1)
            return carry

        lax.fori_loop(0, FULL_ROUNDS // 2, loop_body, 0)
        wait_write(FULL_ROUNDS - 2, 0)
        wait_write(FULL_ROUNDS - 1, 1)

        c_last = wid + NW * FULL_ROUNDS

        @pl.when(c_last < TAIL_C)
        def _():
            fire_loads(FULL_ROUNDS, 0)
            drain_loads(0)
            transpose_block(blk0, tb0)
            fire_write(FULL_ROUNDS, 0)
            wait_write(FULL_ROUNDS, 0)

        @pl.when(c_last == TAIL_C)
        def _():
            pltpu.sync_copy(tail_hbm, tb1.at[pl.ds(0, D)])
            pltpu.sync_copy(
                tb1.at[pl.ds(0, D)], scratch_hbm.at[pl.ds(TAIL_C * CBLK, D)]
            )

    return transpose_kernel


def _build_gather():
    @functools.partial(
        pl.kernel,
        mesh=_mesh,
        out_type=jax.ShapeDtypeStruct((B, S, D), jnp.float32),
        scratch_types=[
            pltpu.VMEM((B_PER_W * 2, HALF), jnp.int32),
            pltpu.VMEM((S, 128), jnp.float32),
            pltpu.VMEM((S, 128), jnp.float32),
            pltpu.SemaphoreType.DMA,
            pltpu.SemaphoreType.DMA,
            pltpu.SemaphoreType.DMA,
            pltpu.SemaphoreType.DMA,
        ],
        compiler_params=pltpu.CompilerParams(use_tc_tiling_on_sc=False),
    )
    def gather_kernel(scratch_hbm, ids_hbm, out_hbm,
                      idsv, rows0, rows1, g0, g1, w0, w1):
        wid = lax.axis_index("s") * NC + lax.axis_index("c")
        b0 = wid * B_PER_W
        rows = (rows0, rows1)
        gsems = (g0, g1)
        wsems = (w0, w1)

        def fire_gather(i, b):
            for h in range(2):
                pltpu.async_copy(
                    scratch_hbm.at[idsv.at[2 * i + h]],
                    rows[b].at[pl.ds(h * HALF, HALF)],
                    gsems[b],
                )

        def wait_gather(i, b):
            for h in range(2):
                pltpu.make_async_copy(
                    scratch_hbm.at[idsv.at[2 * i + h]],
                    rows[b].at[pl.ds(h * HALF, HALF)],
                    gsems[b],
                ).wait()

        def fire_out(i, b):
            pltpu.async_copy(
                rows[b].at[:, pl.ds(0, D)], out_hbm.at[b0 + i], wsems[b]
            )

        def wait_out(i, b):
            pltpu.make_async_copy(
                rows[b].at[:, pl.ds(0, D)], out_hbm.at[b0 + i], wsems[b]
            ).wait()

        pltpu.sync_copy(ids_hbm.at[pl.ds(wid * B_PER_W * 2, B_PER_W * 2)], idsv)

        fire_gather(0, 0)

        def step(i, b):
            @pl.when(i >= 1)
            def _():
                wait_out(i - 1, 1 - b)

            @pl.when(i <= B_PER_W - 2)
            def _():
                fire_gather(i + 1, 1 - b)

            wait_gather(i, b)
            fire_out(i, b)

        def loop_body(j, carry):
            step(2 * j, 0)
            step(2 * j + 1, 1)
            return carry

        lax.fori_loop(0, B_PER_W // 2, loop_body, 0)
        wait_out(B_PER_W - 1, 1)

    return gather_kernel


_transpose = _build_transpose()
_gather = _build_gather()


@jax.jit
def kernel(input_ids, table):
    tableT = jnp.transpose(table)
    ids2d = input_ids.astype(jnp.int32).reshape(B * 2, HALF)
    tail = jnp.pad(table[TAIL_C * CBLK :], ((0, 0), (0, 128 - D)))
    scratch = _transpose(tableT, tail)
    return _gather(scratch, ids2d)


# speedup vs baseline: 1.7959x; 1.7932x over previous
import functools
import jax
import jax.numpy as jnp
from jax import lax
from jax.experimental import pallas as pl
from jax.experimental.pallas import tpu as pltpu
from jax.experimental.pallas import tpu_sc as plsc

VOCAB = 1000000
DIM = 64
BATCH = 4096
SEQ = 200
B = BATCH * SEQ

NC = 2
NS = 16
NW = NC * NS
B_PER_W = B // NW

IDXROW = 128
K = 4
CHUNK = K * IDXROW
N_MACRO = B_PER_W // CHUNK
ROWS_PER_W = B_PER_W // IDXROW


def _build_gather():
    mesh = plsc.VectorSubcoreMesh(core_axis_name="c", subcore_axis_name="s")

    @functools.partial(
        pl.kernel,
        mesh=mesh,
        out_type=jax.ShapeDtypeStruct((B, DIM), jnp.float32),
        scratch_types=[
            pltpu.VMEM((ROWS_PER_W, IDXROW), jnp.int32),
            pltpu.VMEM((CHUNK, DIM), jnp.float32),
            pltpu.VMEM((CHUNK, DIM), jnp.float32),
            pltpu.SemaphoreType.DMA,
            pltpu.SemaphoreType.DMA,
            pltpu.SemaphoreType.DMA,
            pltpu.SemaphoreType.DMA,
        ],
        compiler_params=pltpu.CompilerParams(use_tc_tiling_on_sc=False),
    )
    def gather_kernel(ids_hbm, table_hbm, out_hbm,
                      idx_all, rows0, rows1, g0, g1, o0, o1):
        wid = lax.axis_index("s") * NC + lax.axis_index("c")
        out_base = wid * B_PER_W
        rows = (rows0, rows1)
        gsem = (g0, g1)
        osem = (o0, o1)

        def fire_gather(m, b):
            r0 = m * K
            for j in range(K):
                pltpu.async_copy(
                    table_hbm.at[idx_all.at[r0 + j]],
                    rows[b].at[pl.ds(j * IDXROW, IDXROW)],
                    gsem[b],
                )

        def wait_gather(m, b):
            r0 = m * K
            for j in range(K):
                pltpu.make_async_copy(
                    table_hbm.at[idx_all.at[r0 + j]],
                    rows[b].at[pl.ds(j * IDXROW, IDXROW)],
                    gsem[b],
                ).wait()

        def fire_out(m, b):
            pltpu.async_copy(
                rows[b], out_hbm.at[pl.ds(out_base + m * CHUNK, CHUNK)], osem[b]
            )

        def wait_out(m, b):
            pltpu.make_async_copy(
                rows[b], out_hbm.at[pl.ds(out_base + m * CHUNK, CHUNK)], osem[b]
            ).wait()

        def step(m, b):
            nb = 1 - b
            wait_out(m - 1, nb)
            fire_gather(m + 1, nb)
            wait_gather(m, b)
            fire_out(m, b)

        pltpu.sync_copy(ids_hbm.at[pl.ds(wid * ROWS_PER_W, ROWS_PER_W)], idx_all)

        fire_gather(0, 0)
        fire_gather(1, 1)
        wait_gather(0, 0)
        fire_out(0, 0)

        def loop_body(i, carry):
            m = 1 + 2 * i
            step(m, 1)
            step(m + 1, 0)
            return carry

        lax.fori_loop(0, (N_MACRO - 2) // 2, loop_body, 0)

        wait_out(N_MACRO - 2, 0)
        wait_gather(N_MACRO - 1, 1)
        fire_out(N_MACRO - 1, 1)
        wait_out(N_MACRO - 1, 1)

    return gather_kernel


_gather = _build_gather()


@jax.jit
def kernel(input_ids, table):
    ids2d = input_ids.astype(jnp.int32).reshape(B // IDXROW, IDXROW)
    out = _gather(ids2d, table)
    return out.reshape(BATCH, SEQ, DIM)
